# Initial kernel scaffold; baseline (speedup 1.0000x reference)
import jax
import jax.numpy as jnp
from jax.experimental import pallas as pl


def kernel(x, edge_index, feat_idx, feat_val, W1, W2, fm_embedding, attn_W, attn_b, attn_p, proj_W, proj_b):
    raise NotImplementedError("write your pallas kernel here")


# speedup vs baseline: 3.0548x; 3.0548x over previous
import functools

import jax
import jax.numpy as jnp
from jax import lax
from jax.experimental import pallas as pl
from jax.experimental.pallas import tpu as pltpu
from jax.experimental.pallas import tpu_sc as plsc

N = 10000
E = 320000
D = 128
H = 128
A = 64
K = 8
OUT = 64
V = 129
VP = 256

NC = 2
NS = 16
NW = NC * NS
CHUNK = 128
CHUNKS_PT = -(-E // (NW * CHUNK))
EP = NW * CHUNK * CHUNKS_PT
ROWS_PT = 632
NPAD = NS * ROWS_PT

_SC_MESH = plsc.VectorSubcoreMesh(
    core_axis_name="c", subcore_axis_name="s", num_cores=NC, num_subcores=NS)


def _sc_agg_body(table, edges, zrows, agg_out, idx_s, idx_d, rows, agg_sh, sem):
    c = lax.axis_index("c")
    s = lax.axis_index("s")
    r0 = s * ROWS_PT
    off = c * NPAD + r0

    pltpu.sync_copy(zrows, agg_sh.at[pl.ds(r0, ROWS_PT)])
    plsc.subcore_barrier()

    base = (c * NS + s) * (CHUNK * CHUNKS_PT)

    def body(i, _):
        off_e = base + i * CHUNK
        pltpu.sync_copy(edges.at[pl.ds(off_e, CHUNK)], idx_s)
        pltpu.sync_copy(edges.at[pl.ds(EP + off_e, CHUNK)], idx_d)
        pltpu.async_copy(table.at[idx_s], rows, sem).wait()
        pltpu.async_copy(rows, agg_sh.at[idx_d], sem, add=True).wait()
        return 0

    lax.fori_loop(0, CHUNKS_PT, body, 0)
    plsc.subcore_barrier()

    pltpu.sync_copy(agg_sh.at[pl.ds(r0, ROWS_PT)], agg_out.at[pl.ds(off, ROWS_PT)])


_sc_agg = pl.kernel(
    _sc_agg_body,
    out_type=[jax.ShapeDtypeStruct((NC * NPAD, H), jnp.float32)],
    mesh=_SC_MESH,
    scratch_types=[
        pltpu.VMEM((CHUNK,), jnp.int32),
        pltpu.VMEM((CHUNK,), jnp.int32),
        pltpu.VMEM((CHUNK, H), jnp.float32),
        pltpu.VMEM_SHARED((NPAD, H), jnp.float32),
        pltpu.SemaphoreType.DMA,
    ],
)


def _sc_deg_body(edges, zrows, ones, deg_out, idx_d, ones_v, deg_sh, sem):
    c = lax.axis_index("c")
    s = lax.axis_index("s")
    r0 = s * ROWS_PT
    off = c * NPAD + r0

    pltpu.sync_copy(zrows, deg_sh.at[pl.ds(r0, ROWS_PT)])
    pltpu.sync_copy(ones, ones_v)
    plsc.subcore_barrier()

    base = (c * NS + s) * (CHUNK * CHUNKS_PT)

    def body(i, _):
        off_e = base + i * CHUNK
        pltpu.sync_copy(edges.at[pl.ds(EP + off_e, CHUNK)], idx_d)
        pltpu.async_copy(ones_v, deg_sh.at[idx_d], sem, add=True).wait()
        return 0

    lax.fori_loop(0, CHUNKS_PT, body, 0)
    plsc.subcore_barrier()
    pltpu.sync_copy(deg_sh.at[pl.ds(r0, ROWS_PT)], deg_out.at[pl.ds(off, ROWS_PT)])


_sc_deg = pl.kernel(
    _sc_deg_body,
    out_type=[jax.ShapeDtypeStruct((NC * NPAD, H), jnp.float32)],
    mesh=_SC_MESH,
    scratch_types=[
        pltpu.VMEM((CHUNK,), jnp.int32),
        pltpu.VMEM((CHUNK, H), jnp.float32),
        pltpu.VMEM_SHARED((NPAD, H), jnp.float32),
        pltpu.SemaphoreType.DMA,
    ],
)


BN1 = 1000
BN2 = 400

_IU, _JU = zip(*[(i, j) for i in range(K) for j in range(i + 1, K)])
NPAIR = len(_IU)


def _dense1_body(a0, a1, d0, d1, w, h_out, d_out):
    agg = a0[...] + a1[...]
    deg = jnp.maximum(d0[:, 0:1] + d1[:, 0:1], 1.0)
    h = jnp.dot(agg, w[...], preferred_element_type=jnp.float32) / deg
    h_out[...] = jnp.maximum(h, 0.0)
    d_out[...] = deg


def _dense1(a0, a1, d0, d1, w):
    grid = N // BN1
    return pl.pallas_call(
        _dense1_body,
        grid=(grid,),
        in_specs=[
            pl.BlockSpec((BN1, H), lambda i: (i, 0)),
            pl.BlockSpec((BN1, H), lambda i: (i, 0)),
            pl.BlockSpec((BN1, H), lambda i: (i, 0)),
            pl.BlockSpec((BN1, H), lambda i: (i, 0)),
            pl.BlockSpec((H, H), lambda i: (0, 0)),
        ],
        out_specs=[
            pl.BlockSpec((BN1, H), lambda i: (i, 0)),
            pl.BlockSpec((BN1, 1), lambda i: (i, 0)),
        ],
        out_shape=[
            jax.ShapeDtypeStruct((N, H), jnp.float32),
            jax.ShapeDtypeStruct((N, 1), jnp.float32),
        ],
    )(a0, a1, d0, d1, w)


def _dense2_body(a0, a1, d, w2, fi, fv, fme, aw, ab, ap, pw1, pw2, pb,
                 out_ref):
    agg = a0[...] + a1[...]
    deg = d[...]
    h2 = jnp.maximum(
        jnp.dot(agg, w2[...], preferred_element_type=jnp.float32) / deg, 0.0)

    fii = fi[...]
    fvv = fv[...]
    iota = lax.broadcasted_iota(jnp.int32, (BN2, VP), 1)
    embs = []
    for k in range(K):
        oh = jnp.where(iota == fii[:, k:k + 1], fvv[:, k:k + 1], 0.0)
        embs.append(jnp.dot(oh, fme[...], preferred_element_type=jnp.float32))

    aww = aw[...]
    abb = ab[...]
    app = ap[...]
    scores = []
    for i, j in zip(_IU, _JU):
        pij = embs[i] * embs[j]
        att = jnp.maximum(
            jnp.dot(pij, aww, preferred_element_type=jnp.float32) + abb, 0.0)
        scores.append(jnp.sum(att * app, axis=1, keepdims=True))
    sc = jnp.concatenate(scores, axis=1)
    m = jnp.max(sc, axis=1, keepdims=True)
    e = jnp.exp(sc - m)
    alpha = e / jnp.sum(e, axis=1, keepdims=True)

    afm = jnp.zeros((BN2, H), jnp.float32)
    for p, (i, j) in enumerate(zip(_IU, _JU)):
        afm = afm + alpha[:, p:p + 1] * (embs[i] * embs[j])

    out = (jnp.dot(h2, pw1[...], preferred_element_type=jnp.float32)
           + jnp.dot(afm, pw2[...], preferred_element_type=jnp.float32)
           + pb[...])
    out_ref[...] = out


def _dense2(a0, a1, d, w2, fi, fv, fme, aw, ab, ap, pw1, pw2, pb):
    grid = N // BN2
    row = lambda i: (i, 0)
    full = lambda i: (0, 0)
    return pl.pallas_call(
        _dense2_body,
        grid=(grid,),
        in_specs=[
            pl.BlockSpec((BN2, H), row),
            pl.BlockSpec((BN2, H), row),
            pl.BlockSpec((BN2, 1), row),
            pl.BlockSpec((H, H), full),
            pl.BlockSpec((BN2, K), row),
            pl.BlockSpec((BN2, K), row),
            pl.BlockSpec((VP, H), full),
            pl.BlockSpec((H, A), full),
            pl.BlockSpec((1, A), full),
            pl.BlockSpec((1, A), full),
            pl.BlockSpec((H, OUT), full),
            pl.BlockSpec((H, OUT), full),
            pl.BlockSpec((1, OUT), full),
        ],
        out_specs=pl.BlockSpec((BN2, OUT), row),
        out_shape=jax.ShapeDtypeStruct((N, OUT), jnp.float32),
    )(a0, a1, d, w2, fi, fv, fme, aw, ab, ap, pw1, pw2, pb)


def kernel(x, edge_index, feat_idx, feat_val, W1, W2, fm_embedding,
           attn_W, attn_b, attn_p, proj_W, proj_b):
    src = edge_index[0]
    dst = edge_index[1]
    pad = EP - E
    edges = jnp.concatenate([
        src, jnp.zeros((pad,), src.dtype),
        dst, jnp.full((pad,), N, dst.dtype),
    ])

    zrows = jnp.zeros((ROWS_PT, H), jnp.float32)
    ones128 = jnp.ones((CHUNK, H), jnp.float32)

    (aggp1,) = _sc_agg(x, edges, zrows)
    (degp,) = _sc_deg(edges, zrows, ones128)

    h1, dcol = _dense1(aggp1[:N], aggp1[NPAD:NPAD + N],
                       degp[:N], degp[NPAD:NPAD + N], W1)

    (aggp2,) = _sc_agg(h1, edges, zrows)

    fme = jnp.concatenate(
        [fm_embedding, jnp.zeros((VP - V, H), jnp.float32)], axis=0)
    out = _dense2(aggp2[:N], aggp2[NPAD:NPAD + N], dcol, W2,
                  feat_idx, feat_val, fme,
                  attn_W, attn_b, attn_p.reshape(1, A),
                  proj_W[:H], proj_W[H:], proj_b.reshape(1, OUT))
    return out
